# baseline (speedup 1.0000x reference)
import functools

import jax
import jax.numpy as jnp
from jax import lax
from jax.experimental import pallas as pl
from jax.experimental.pallas import tpu as pltpu
from jax.experimental.pallas import tpu_sc as plsc

N = 10000
E = 320000
D = 128
H = 128
Z = 64

NC = 2
NS = 16
NW = NC * NS

CL = 125
CHUNKS = 80
G = 8
NPAN = CHUNKS // G
EPW = CHUNKS * CL

SLAB = 632
N_ACC = NS * SLAB
N_HIST = 10240

RB = 400
GRID_R = N // RB

@functools.cache
def _mesh():
    return plsc.VectorSubcoreMesh(core_axis_name="c", subcore_axis_name="s",
                                  num_cores=NC, num_subcores=NS)


def _deg_body(dst_hbm, out_hbm, idx_v, hist_v):
    c = lax.axis_index("c")
    s = lax.axis_index("s")
    w = c * NS + s
    zeros16 = jnp.zeros((16,), jnp.float32)
    ones16 = jnp.ones((16,), jnp.float32)

    def _zero(m, _):
        hist_v[pl.ds(m * 16, 16)] = zeros16
        return _

    lax.fori_loop(0, N_HIST // 16, _zero, None)
    pltpu.sync_copy(dst_hbm.at[w], idx_v)

    def _accum(m, _):
        idx16 = idx_v[pl.ds(m * 16, 16)]
        plsc.addupdate_scatter(hist_v, [idx16], ones16)
        return _

    lax.fori_loop(0, EPW // 16, _accum, None)
    pltpu.sync_copy(hist_v, out_hbm.at[w])


@functools.cache
def _deg_kernel():
    return pl.kernel(
        _deg_body,
        out_type=jax.ShapeDtypeStruct((NW, N_HIST), jnp.float32),
        mesh=_mesh(),
        compiler_params=pltpu.CompilerParams(needs_layout_passes=False),
        scratch_types=[
            pltpu.VMEM((EPW,), jnp.int32),
            pltpu.VMEM((N_HIST,), jnp.float32),
        ],
    )


def _prop_body(y_hbm, src_hbm, dst_hbm, zero_hbm, out_hbm,
               src_p, dst_v, rows_v, rsem, isem, acc_sh):
    c = lax.axis_index("c")
    s = lax.axis_index("s")
    pltpu.sync_copy(zero_hbm, acc_sh.at[pl.ds(s * SLAB, SLAB)])
    pltpu.sync_copy(dst_hbm.at[c, s], dst_v)
    pltpu.sync_copy(src_hbm.at[c, s, 0], src_p.at[0])
    plsc.subcore_barrier()

    def _gather(j, slot):
        pltpu.async_copy(y_hbm.at[src_p.at[(j // G) % 2, j % G]],
                         rows_v.at[slot], rsem.at[slot])

    def _gather_wait(j, slot):
        pltpu.make_async_copy(y_hbm.at[src_p.at[(j // G) % 2, j % G]],
                              rows_v.at[slot], rsem.at[slot]).wait()

    _gather(0, 0)
    _gather(1, 1)

    def _chunk(j, _):
        b = j % 2
        pan = j // G
        off = j % G

        @pl.when(jnp.logical_and(off == 0, pan + 1 < NPAN))
        def _():
            pltpu.async_copy(src_hbm.at[c, s, pan + 1],
                             src_p.at[(pan + 1) % 2], isem.at[(pan + 1) % 2])

        @pl.when(jnp.logical_and(off == G - 2, pan + 1 < NPAN))
        def _():
            pltpu.make_async_copy(src_hbm.at[c, s, pan + 1],
                                  src_p.at[(pan + 1) % 2],
                                  isem.at[(pan + 1) % 2]).wait()

        _gather_wait(j, b)
        pltpu.sync_copy(rows_v.at[b], acc_sh.at[dst_v.at[j]], add=True)

        @pl.when(j + 2 < CHUNKS)
        def _():
            _gather(j + 2, b)

        return _

    lax.fori_loop(0, CHUNKS, _chunk, None)
    plsc.subcore_barrier()
    pltpu.sync_copy(acc_sh.at[pl.ds(s * SLAB, SLAB)],
                    out_hbm.at[c, pl.ds(s * SLAB, SLAB)])


@functools.cache
def _prop_kernel():
    return pl.kernel(
        _prop_body,
        out_type=jax.ShapeDtypeStruct((NC, N_ACC, D), jnp.float32),
        mesh=_mesh(),
        compiler_params=pltpu.CompilerParams(needs_layout_passes=False),
        scratch_types=[
            pltpu.VMEM((2, G, CL), jnp.int32),
            pltpu.VMEM((CHUNKS, CL), jnp.int32),
            pltpu.VMEM((2, CL, D), jnp.float32),
            pltpu.SemaphoreType.DMA((2,)),
            pltpu.SemaphoreType.DMA((2,)),
            pltpu.VMEM_SHARED((N_ACC, D), jnp.float32),
        ],
    )


def _dinv_body(hist_ref, out_ref):
    deg = jnp.sum(hist_ref[...], axis=0) + 1.0
    out_ref[...] = lax.rsqrt(deg)


def _xw_body(x_ref, w_ref, out_ref):
    out_ref[...] = jnp.dot(x_ref[...], w_ref[...],
                           preferred_element_type=jnp.float32)


def _scale_body(xw_ref, dinv_ref, out_ref):
    out_ref[...] = xw_ref[...] * dinv_ref[...]


def _sem_body(h_ref, w_ref, b_ref, s_ref, sum_ref, sq_ref):
    s = jnp.dot(h_ref[...], w_ref[...],
                preferred_element_type=jnp.float32) + b_ref[...]
    s_ref[...] = s
    ss = jnp.sum(s, axis=0, keepdims=True)
    qs = jnp.sum(s * s, axis=0, keepdims=True)
    i = pl.program_id(0)

    @pl.when(i == 0)
    def _init():
        sum_ref[...] = ss
        sq_ref[...] = qs

    @pl.when(i != 0)
    def _acc():
        sum_ref[...] += ss
        sq_ref[...] += qs


def _mid_body(p_ref, y_ref, dinv_ref, b1_ref, w2_ref, out_ref):
    h1 = jax.nn.relu(dinv_ref[...] * (p_ref[0] + p_ref[1] + y_ref[...])
                     + b1_ref[...])
    out_ref[...] = jnp.dot(h1, w2_ref[...],
                           preferred_element_type=jnp.float32) * dinv_ref[...]


def _heads_body(p_ref, y_ref, dinv_ref, b2_ref, wt1_ref, bt1_ref,
                t_ref, sum_t_ref, sq_t_ref):
    h2 = dinv_ref[...] * (p_ref[0] + p_ref[1] + y_ref[...]) + b2_ref[...]
    t = jnp.dot(h2, wt1_ref[...], preferred_element_type=jnp.float32) \
        + bt1_ref[...]
    t_ref[...] = t
    st = jnp.sum(t, axis=0, keepdims=True)
    qt = jnp.sum(t * t, axis=0, keepdims=True)
    i = pl.program_id(0)

    @pl.when(i == 0)
    def _init():
        sum_t_ref[...] = st
        sq_t_ref[...] = qt

    @pl.when(i != 0)
    def _acc():
        sum_t_ref[...] += st
        sq_t_ref[...] += qt


def _final_body(t_ref, s_ref, sum_t_ref, sq_t_ref, sum_s_ref, sq_s_ref,
                gt_ref, btb_ref, wt2_ref, bt2_ref,
                gs_ref, bsb_ref, ws2_ref, bs2_ref,
                zt_ref, zs_ref):
    inv_n = 1.0 / N

    def _branch(x, sum_ref, sq_ref, g_ref, beta_ref, w_ref, b_ref):
        m = sum_ref[...] * inv_n
        v = sq_ref[...] * inv_n - m * m
        xn = jax.nn.relu((x - m) * lax.rsqrt(v + 1e-5) * g_ref[...]
                         + beta_ref[...])
        z = jnp.dot(xn, w_ref[...], preferred_element_type=jnp.float32) \
            + b_ref[...]
        nrm = jnp.sqrt(jnp.sum(z * z, axis=1, keepdims=True))
        return z / jnp.maximum(nrm, 1e-12)

    zt_ref[...] = _branch(t_ref[...], sum_t_ref, sq_t_ref,
                          gt_ref, btb_ref, wt2_ref, bt2_ref)
    zs_ref[...] = _branch(s_ref[...], sum_s_ref, sq_s_ref,
                          gs_ref, bsb_ref, ws2_ref, bs2_ref)


def _full(shape):
    return pl.BlockSpec(shape, lambda i: (0,) * len(shape))


def _rows(shape):
    return pl.BlockSpec(shape, lambda i: (i,) + (0,) * (len(shape) - 1))


_f32 = jnp.float32


def _dinv_call(hists3d):
    return pl.pallas_call(
        _dinv_body,
        out_shape=jax.ShapeDtypeStruct((N_HIST // 128, 128), _f32),
    )(hists3d)


def _xw_call(x, w1):
    return pl.pallas_call(
        _xw_body,
        grid=(GRID_R,),
        in_specs=[_rows((RB, D)), _full((D, H))],
        out_specs=_rows((RB, H)),
        out_shape=jax.ShapeDtypeStruct((N, H), _f32),
    )(x, w1)


def _scale_call(xw, dinv_col):
    return pl.pallas_call(
        _scale_body,
        grid=(GRID_R,),
        in_specs=[_rows((RB, H)), _rows((RB, 1))],
        out_specs=_rows((RB, H)),
        out_shape=jax.ShapeDtypeStruct((N, H), _f32),
    )(xw, dinv_col)


def _sem_call(h_sem, ws1, bs1r):
    return pl.pallas_call(
        _sem_body,
        grid=(GRID_R,),
        in_specs=[_rows((RB, D)), _full((D, H)), _full((1, H))],
        out_specs=[_rows((RB, H)), _full((1, H)), _full((1, H))],
        out_shape=[jax.ShapeDtypeStruct((N, H), _f32),
                   jax.ShapeDtypeStruct((1, H), _f32),
                   jax.ShapeDtypeStruct((1, H), _f32)],
    )(h_sem, ws1, bs1r)


def _mid_call(p, y1, dinv_col, b1r, w2):
    return pl.pallas_call(
        _mid_body,
        grid=(GRID_R,),
        in_specs=[pl.BlockSpec((NC, RB, H), lambda i: (0, i, 0)),
                  _rows((RB, H)), _rows((RB, 1)), _full((1, H)),
                  _full((H, H))],
        out_specs=_rows((RB, H)),
        out_shape=jax.ShapeDtypeStruct((N, H), _f32),
    )(p, y1, dinv_col, b1r, w2)


def _heads_call(p, y2, dinv_col, b2r, wt1, bt1r):
    return pl.pallas_call(
        _heads_body,
        grid=(GRID_R,),
        in_specs=[pl.BlockSpec((NC, RB, H), lambda i: (0, i, 0)),
                  _rows((RB, H)), _rows((RB, 1)), _full((1, H)),
                  _full((H, H)), _full((1, H))],
        out_specs=[_rows((RB, H)),
                   _full((1, H)), _full((1, H))],
        out_shape=[jax.ShapeDtypeStruct((N, H), _f32),
                   jax.ShapeDtypeStruct((1, H), _f32),
                   jax.ShapeDtypeStruct((1, H), _f32)],
    )(p, y2, dinv_col, b2r, wt1, bt1r)


def _final_call(t, s, st, qt, ss, qs, gtr, btbr, wt2, bt2r,
                gsr, bsbr, ws2, bs2r):
    return pl.pallas_call(
        _final_body,
        grid=(GRID_R,),
        in_specs=[_rows((RB, H)), _rows((RB, H)),
                  _full((1, H)), _full((1, H)), _full((1, H)), _full((1, H)),
                  _full((1, H)), _full((1, H)), _full((H, Z)), _full((1, Z)),
                  _full((1, H)), _full((1, H)), _full((H, Z)), _full((1, Z))],
        out_specs=[_rows((RB, Z)), _rows((RB, Z))],
        out_shape=[jax.ShapeDtypeStruct((N, Z), _f32),
                   jax.ShapeDtypeStruct((N, Z), _f32)],
    )(t, s, st, qt, ss, qs, gtr, btbr, wt2, bt2r, gsr, bsbr, ws2, bs2r)


def kernel(x_topo, edge_index, h_sem, W1, b1, W2, b2, Wt1, bt1, gt, btb,
           Wt2, bt2, Ws1, bs1, gs, bsb, Ws2, bs2):
    src_idx = edge_index[0].reshape(NC, NS, NPAN, G, CL)
    dst_idx = edge_index[1].reshape(NC, NS, CHUNKS, CL)
    dst_flat = edge_index[1].reshape(NW, EPW)
    zero_slab = jnp.zeros((SLAB, D), _f32)

    hists = _deg_kernel()(dst_flat)
    dinv = _dinv_call(hists.reshape(NW, N_HIST // 128, 128))
    dinv_col = dinv.reshape(N_HIST, 1)[:N]

    b1r, b2r = b1.reshape(1, H), b2.reshape(1, H)
    bt1r, bs1r = bt1.reshape(1, H), bs1.reshape(1, H)
    gtr, btbr = gt.reshape(1, H), btb.reshape(1, H)
    gsr, bsbr = gs.reshape(1, H), bsb.reshape(1, H)
    bt2r, bs2r = bt2.reshape(1, Z), bs2.reshape(1, Z)

    xw = _xw_call(x_topo, W1)
    y1 = _scale_call(xw, dinv_col)
    p1 = _prop_kernel()(y1, src_idx, dst_idx, zero_slab)
    s, ss, qs = _sem_call(h_sem, Ws1, bs1r)
    y2 = _mid_call(p1, y1, dinv_col, b1r, W2)
    p2 = _prop_kernel()(y2, src_idx, dst_idx, zero_slab)
    t, st, qt = _heads_call(p2, y2, dinv_col, b2r, Wt1, bt1r)
    z_topo, z_sem = _final_call(t, s, st, qt, ss, qs, gtr, btbr, Wt2, bt2r,
                                gsr, bsbr, Ws2, bs2r)
    return (z_topo, z_sem)


# speedup vs baseline: 1.0408x; 1.0408x over previous
import functools

import jax
import jax.numpy as jnp
from jax import lax
from jax.experimental import pallas as pl
from jax.experimental.pallas import tpu as pltpu
from jax.experimental.pallas import tpu_sc as plsc

N = 10000
E = 320000
D = 128
H = 128
Z = 64

NC = 2
NS = 16
NW = NC * NS

CL = 125
CHUNKS = 80
G = 8
NPAN = CHUNKS // G
EPW = CHUNKS * CL

SLAB = 632
N_ACC = NS * SLAB
N_HIST = 10240

RB = 400
GRID_R = N // RB

@functools.cache
def _mesh():
    return plsc.VectorSubcoreMesh(core_axis_name="c", subcore_axis_name="s",
                                  num_cores=NC, num_subcores=NS)


def _deg_body(dst_hbm, out_hbm, idx_v, hist_v):
    c = lax.axis_index("c")
    s = lax.axis_index("s")
    w = c * NS + s
    zeros16 = jnp.zeros((16,), jnp.float32)
    ones16 = jnp.ones((16,), jnp.float32)

    def _zero(m, _):
        hist_v[pl.ds(m * 16, 16)] = zeros16
        return _

    lax.fori_loop(0, N_HIST // 16, _zero, None)
    pltpu.sync_copy(dst_hbm.at[w], idx_v)

    def _accum(m, _):
        idx16 = idx_v[pl.ds(m * 16, 16)]
        plsc.addupdate_scatter(hist_v, [idx16], ones16)
        return _

    lax.fori_loop(0, EPW // 16, _accum, None)
    pltpu.sync_copy(hist_v, out_hbm.at[w])


@functools.cache
def _deg_kernel():
    return pl.kernel(
        _deg_body,
        out_type=jax.ShapeDtypeStruct((NW, N_HIST), jnp.float32),
        mesh=_mesh(),
        compiler_params=pltpu.CompilerParams(needs_layout_passes=False),
        scratch_types=[
            pltpu.VMEM((EPW,), jnp.int32),
            pltpu.VMEM((N_HIST,), jnp.float32),
        ],
    )


def _prop_body(y_hbm, src_hbm, dst_hbm, zero_hbm, out_hbm,
               src_p, dst_v, rows_v, rsem, isem, acc_sh):
    c = lax.axis_index("c")
    s = lax.axis_index("s")
    pltpu.sync_copy(zero_hbm, acc_sh.at[pl.ds(s * SLAB, SLAB)])
    pltpu.sync_copy(dst_hbm.at[c, s], dst_v)
    pltpu.sync_copy(src_hbm.at[c, s, 0], src_p.at[0])
    plsc.subcore_barrier()

    def _gather(j, slot):
        pltpu.async_copy(y_hbm.at[src_p.at[(j // G) % 2, j % G]],
                         rows_v.at[slot], rsem.at[slot])

    def _gather_wait(j, slot):
        pltpu.make_async_copy(y_hbm.at[src_p.at[(j // G) % 2, j % G]],
                              rows_v.at[slot], rsem.at[slot]).wait()

    _gather(0, 0)
    _gather(1, 1)

    def _chunk(j, _):
        b = j % 2
        pan = j // G
        off = j % G

        @pl.when(jnp.logical_and(off == 0, pan + 1 < NPAN))
        def _():
            pltpu.async_copy(src_hbm.at[c, s, pan + 1],
                             src_p.at[(pan + 1) % 2], isem.at[(pan + 1) % 2])

        @pl.when(jnp.logical_and(off == G - 2, pan + 1 < NPAN))
        def _():
            pltpu.make_async_copy(src_hbm.at[c, s, pan + 1],
                                  src_p.at[(pan + 1) % 2],
                                  isem.at[(pan + 1) % 2]).wait()

        _gather_wait(j, b)
        pltpu.sync_copy(rows_v.at[b], acc_sh.at[dst_v.at[j]], add=True)

        @pl.when(j + 2 < CHUNKS)
        def _():
            _gather(j + 2, b)

        return _

    lax.fori_loop(0, CHUNKS, _chunk, None)
    plsc.subcore_barrier()
    pltpu.sync_copy(acc_sh.at[pl.ds(s * SLAB, SLAB)],
                    out_hbm.at[c, pl.ds(s * SLAB, SLAB)])


@functools.cache
def _prop_kernel():
    return pl.kernel(
        _prop_body,
        out_type=jax.ShapeDtypeStruct((NC, N_ACC, D), jnp.float32),
        mesh=_mesh(),
        compiler_params=pltpu.CompilerParams(needs_layout_passes=False),
        scratch_types=[
            pltpu.VMEM((2, G, CL), jnp.int32),
            pltpu.VMEM((CHUNKS, CL), jnp.int32),
            pltpu.VMEM((2, CL, D), jnp.float32),
            pltpu.SemaphoreType.DMA((2,)),
            pltpu.SemaphoreType.DMA((2,)),
            pltpu.VMEM_SHARED((N_ACC, D), jnp.float32),
        ],
    )


def _dinv_body(hist_ref, out_ref):
    deg = jnp.sum(hist_ref[...], axis=0) + 1.0
    out_ref[...] = lax.rsqrt(deg)


def _y1_body(x_ref, w_ref, dinv_ref, out_ref):
    out_ref[...] = jnp.dot(x_ref[...], w_ref[...],
                           preferred_element_type=jnp.float32) * dinv_ref[...]


def _sem_body(h_ref, w_ref, b_ref, s_ref, sum_ref, sq_ref):
    s = jnp.dot(h_ref[...], w_ref[...],
                preferred_element_type=jnp.float32) + b_ref[...]
    s_ref[...] = s
    ss = jnp.sum(s, axis=0, keepdims=True)
    qs = jnp.sum(s * s, axis=0, keepdims=True)
    i = pl.program_id(0)

    @pl.when(i == 0)
    def _init():
        sum_ref[...] = ss
        sq_ref[...] = qs

    @pl.when(i != 0)
    def _acc():
        sum_ref[...] += ss
        sq_ref[...] += qs


def _mid_body(p_ref, y_ref, dinv_ref, b1_ref, w2_ref, out_ref):
    h1 = jax.nn.relu(dinv_ref[...] * (p_ref[0] + p_ref[1] + y_ref[...])
                     + b1_ref[...])
    out_ref[...] = jnp.dot(h1, w2_ref[...],
                           preferred_element_type=jnp.float32) * dinv_ref[...]


def _heads_body(p_ref, y_ref, dinv_ref, b2_ref, wt1_ref, bt1_ref,
                t_ref, sum_t_ref, sq_t_ref):
    h2 = dinv_ref[...] * (p_ref[0] + p_ref[1] + y_ref[...]) + b2_ref[...]
    t = jnp.dot(h2, wt1_ref[...], preferred_element_type=jnp.float32) \
        + bt1_ref[...]
    t_ref[...] = t
    st = jnp.sum(t, axis=0, keepdims=True)
    qt = jnp.sum(t * t, axis=0, keepdims=True)
    i = pl.program_id(0)

    @pl.when(i == 0)
    def _init():
        sum_t_ref[...] = st
        sq_t_ref[...] = qt

    @pl.when(i != 0)
    def _acc():
        sum_t_ref[...] += st
        sq_t_ref[...] += qt


def _final_body(t_ref, s_ref, sum_t_ref, sq_t_ref, sum_s_ref, sq_s_ref,
                gt_ref, btb_ref, wt2_ref, bt2_ref,
                gs_ref, bsb_ref, ws2_ref, bs2_ref,
                zt_ref, zs_ref):
    inv_n = 1.0 / N

    def _branch(x, sum_ref, sq_ref, g_ref, beta_ref, w_ref, b_ref):
        m = sum_ref[...] * inv_n
        v = sq_ref[...] * inv_n - m * m
        xn = jax.nn.relu((x - m) * lax.rsqrt(v + 1e-5) * g_ref[...]
                         + beta_ref[...])
        z = jnp.dot(xn, w_ref[...], preferred_element_type=jnp.float32) \
            + b_ref[...]
        nrm = jnp.sqrt(jnp.sum(z * z, axis=1, keepdims=True))
        return z / jnp.maximum(nrm, 1e-12)

    zt_ref[...] = _branch(t_ref[...], sum_t_ref, sq_t_ref,
                          gt_ref, btb_ref, wt2_ref, bt2_ref)
    zs_ref[...] = _branch(s_ref[...], sum_s_ref, sq_s_ref,
                          gs_ref, bsb_ref, ws2_ref, bs2_ref)


def _full(shape):
    return pl.BlockSpec(shape, lambda i: (0,) * len(shape))


def _rows(shape):
    return pl.BlockSpec(shape, lambda i: (i,) + (0,) * (len(shape) - 1))


_f32 = jnp.float32


def _dinv_call(hists3d):
    return pl.pallas_call(
        _dinv_body,
        out_shape=jax.ShapeDtypeStruct((N_HIST // 128, 128), _f32),
    )(hists3d)


def _y1_call(x, w1, dinv_col):
    return pl.pallas_call(
        _y1_body,
        grid=(GRID_R,),
        in_specs=[_rows((RB, D)), _full((D, H)), _rows((RB, 1))],
        out_specs=_rows((RB, H)),
        out_shape=jax.ShapeDtypeStruct((N, H), _f32),
    )(x, w1, dinv_col)


def _sem_call(h_sem, ws1, bs1r):
    return pl.pallas_call(
        _sem_body,
        grid=(GRID_R,),
        in_specs=[_rows((RB, D)), _full((D, H)), _full((1, H))],
        out_specs=[_rows((RB, H)), _full((1, H)), _full((1, H))],
        out_shape=[jax.ShapeDtypeStruct((N, H), _f32),
                   jax.ShapeDtypeStruct((1, H), _f32),
                   jax.ShapeDtypeStruct((1, H), _f32)],
    )(h_sem, ws1, bs1r)


def _mid_call(p, y1, dinv_col, b1r, w2):
    return pl.pallas_call(
        _mid_body,
        grid=(GRID_R,),
        in_specs=[pl.BlockSpec((NC, RB, H), lambda i: (0, i, 0)),
                  _rows((RB, H)), _rows((RB, 1)), _full((1, H)),
                  _full((H, H))],
        out_specs=_rows((RB, H)),
        out_shape=jax.ShapeDtypeStruct((N, H), _f32),
    )(p, y1, dinv_col, b1r, w2)


def _heads_call(p, y2, dinv_col, b2r, wt1, bt1r):
    return pl.pallas_call(
        _heads_body,
        grid=(GRID_R,),
        in_specs=[pl.BlockSpec((NC, RB, H), lambda i: (0, i, 0)),
                  _rows((RB, H)), _rows((RB, 1)), _full((1, H)),
                  _full((H, H)), _full((1, H))],
        out_specs=[_rows((RB, H)),
                   _full((1, H)), _full((1, H))],
        out_shape=[jax.ShapeDtypeStruct((N, H), _f32),
                   jax.ShapeDtypeStruct((1, H), _f32),
                   jax.ShapeDtypeStruct((1, H), _f32)],
    )(p, y2, dinv_col, b2r, wt1, bt1r)


def _final_call(t, s, st, qt, ss, qs, gtr, btbr, wt2, bt2r,
                gsr, bsbr, ws2, bs2r):
    return pl.pallas_call(
        _final_body,
        grid=(GRID_R,),
        in_specs=[_rows((RB, H)), _rows((RB, H)),
                  _full((1, H)), _full((1, H)), _full((1, H)), _full((1, H)),
                  _full((1, H)), _full((1, H)), _full((H, Z)), _full((1, Z)),
                  _full((1, H)), _full((1, H)), _full((H, Z)), _full((1, Z))],
        out_specs=[_rows((RB, Z)), _rows((RB, Z))],
        out_shape=[jax.ShapeDtypeStruct((N, Z), _f32),
                   jax.ShapeDtypeStruct((N, Z), _f32)],
    )(t, s, st, qt, ss, qs, gtr, btbr, wt2, bt2r, gsr, bsbr, ws2, bs2r)


def kernel(x_topo, edge_index, h_sem, W1, b1, W2, b2, Wt1, bt1, gt, btb,
           Wt2, bt2, Ws1, bs1, gs, bsb, Ws2, bs2):
    src_idx = edge_index[0].reshape(NC, NS, NPAN, G, CL)
    dst_idx = edge_index[1].reshape(NC, NS, CHUNKS, CL)
    dst_flat = edge_index[1].reshape(NW, EPW)
    zero_slab = jnp.zeros((SLAB, D), _f32)

    hists = _deg_kernel()(dst_flat)
    dinv = _dinv_call(hists.reshape(NW, N_HIST // 128, 128))
    dinv_col = dinv.reshape(N_HIST, 1)[:N]

    b1r, b2r = b1.reshape(1, H), b2.reshape(1, H)
    bt1r, bs1r = bt1.reshape(1, H), bs1.reshape(1, H)
    gtr, btbr = gt.reshape(1, H), btb.reshape(1, H)
    gsr, bsbr = gs.reshape(1, H), bsb.reshape(1, H)
    bt2r, bs2r = bt2.reshape(1, Z), bs2.reshape(1, Z)

    y1 = _y1_call(x_topo, W1, dinv_col)
    p1 = _prop_kernel()(y1, src_idx, dst_idx, zero_slab)
    s, ss, qs = _sem_call(h_sem, Ws1, bs1r)
    y2 = _mid_call(p1, y1, dinv_col, b1r, W2)
    p2 = _prop_kernel()(y2, src_idx, dst_idx, zero_slab)
    t, st, qt = _heads_call(p2, y2, dinv_col, b2r, Wt1, bt1r)
    z_topo, z_sem = _final_call(t, s, st, qt, ss, qs, gtr, btbr, Wt2, bt2r,
                                gsr, bsbr, Ws2, bs2r)
    return (z_topo, z_sem)


# speedup vs baseline: 1.0424x; 1.0016x over previous
import functools

import jax
import jax.numpy as jnp
from jax import lax
from jax.experimental import pallas as pl
from jax.experimental.pallas import tpu as pltpu
from jax.experimental.pallas import tpu_sc as plsc

N = 10000
E = 320000
D = 128
H = 128
Z = 64

NC = 2
NS = 16
NW = NC * NS

CL = 125
CHUNKS = 80
G = 8
NPAN = CHUNKS // G
EPW = CHUNKS * CL

SLAB = 632
N_ACC = NS * SLAB
N_HIST = 10240

RB = 400
GRID_R = N // RB

@functools.cache
def _mesh():
    return plsc.VectorSubcoreMesh(core_axis_name="c", subcore_axis_name="s",
                                  num_cores=NC, num_subcores=NS)


def _deg_body(dst_hbm, out_hbm, idx_v, hist_v):
    c = lax.axis_index("c")
    s = lax.axis_index("s")
    w = c * NS + s
    zeros16 = jnp.zeros((16,), jnp.float32)
    ones16 = jnp.ones((16,), jnp.float32)

    def _zero(m, _):
        hist_v[pl.ds(m * 16, 16)] = zeros16
        return _

    lax.fori_loop(0, N_HIST // 16, _zero, None)
    pltpu.sync_copy(dst_hbm.at[w], idx_v)

    def _accum(m, _):
        idx16 = idx_v[pl.ds(m * 16, 16)]
        plsc.addupdate_scatter(hist_v, [idx16], ones16)
        return _

    lax.fori_loop(0, EPW // 16, _accum, None)
    pltpu.sync_copy(hist_v, out_hbm.at[w])


@functools.cache
def _deg_kernel():
    return pl.kernel(
        _deg_body,
        out_type=jax.ShapeDtypeStruct((NW, N_HIST), jnp.float32),
        mesh=_mesh(),
        compiler_params=pltpu.CompilerParams(needs_layout_passes=False),
        scratch_types=[
            pltpu.VMEM((EPW,), jnp.int32),
            pltpu.VMEM((N_HIST,), jnp.float32),
        ],
    )


def _prop_body(y_hbm, src_hbm, dst_hbm, zero_hbm, out_hbm,
               src_p, dst_v, rows_v, rsem, isem, acc_sh):
    c = lax.axis_index("c")
    s = lax.axis_index("s")
    pltpu.sync_copy(zero_hbm, acc_sh.at[pl.ds(s * SLAB, SLAB)])
    pltpu.sync_copy(dst_hbm.at[c, s], dst_v)
    pltpu.sync_copy(src_hbm.at[c, s, 0], src_p.at[0])
    plsc.subcore_barrier()

    def _gather(j, slot):
        pltpu.async_copy(y_hbm.at[src_p.at[(j // G) % 2, j % G]],
                         rows_v.at[slot], rsem.at[slot])

    def _gather_wait(j, slot):
        pltpu.make_async_copy(y_hbm.at[src_p.at[(j // G) % 2, j % G]],
                              rows_v.at[slot], rsem.at[slot]).wait()

    _gather(0, 0)
    _gather(1, 1)

    def _chunk(j, _):
        b = j % 2
        pan = j // G
        off = j % G

        @pl.when(jnp.logical_and(off == 0, pan + 1 < NPAN))
        def _():
            pltpu.async_copy(src_hbm.at[c, s, pan + 1],
                             src_p.at[(pan + 1) % 2], isem.at[(pan + 1) % 2])

        @pl.when(jnp.logical_and(off == G - 2, pan + 1 < NPAN))
        def _():
            pltpu.make_async_copy(src_hbm.at[c, s, pan + 1],
                                  src_p.at[(pan + 1) % 2],
                                  isem.at[(pan + 1) % 2]).wait()

        _gather_wait(j, b)
        pltpu.sync_copy(rows_v.at[b], acc_sh.at[dst_v.at[j]], add=True)

        @pl.when(j + 2 < CHUNKS)
        def _():
            _gather(j + 2, b)

        return _

    lax.fori_loop(0, CHUNKS, _chunk, None)
    plsc.subcore_barrier()
    pltpu.sync_copy(acc_sh.at[pl.ds(s * SLAB, SLAB)],
                    out_hbm.at[c, pl.ds(s * SLAB, SLAB)])


@functools.cache
def _prop_kernel():
    return pl.kernel(
        _prop_body,
        out_type=jax.ShapeDtypeStruct((NC, N_ACC, D), jnp.float32),
        mesh=_mesh(),
        compiler_params=pltpu.CompilerParams(needs_layout_passes=False),
        scratch_types=[
            pltpu.VMEM((2, G, CL), jnp.int32),
            pltpu.VMEM((CHUNKS, CL), jnp.int32),
            pltpu.VMEM((2, CL, D), jnp.float32),
            pltpu.SemaphoreType.DMA((2,)),
            pltpu.SemaphoreType.DMA((2,)),
            pltpu.VMEM_SHARED((N_ACC, D), jnp.float32),
        ],
    )


def _dinv_body(hist_ref, out_ref):
    deg = jnp.sum(hist_ref[...], axis=0) + 1.0
    out_ref[...] = lax.rsqrt(deg)


def _y1_body(x_ref, w_ref, dinv_ref, out_ref):
    out_ref[...] = jnp.dot(x_ref[...], w_ref[...],
                           preferred_element_type=jnp.float32) * dinv_ref[...]


def _sem_body(h_ref, w_ref, b_ref, s_ref, sum_ref, sq_ref):
    s = jnp.dot(h_ref[...], w_ref[...],
                preferred_element_type=jnp.float32) + b_ref[...]
    s_ref[...] = s
    ss = jnp.sum(s, axis=0, keepdims=True)
    qs = jnp.sum(s * s, axis=0, keepdims=True)
    i = pl.program_id(0)

    @pl.when(i == 0)
    def _init():
        sum_ref[...] = ss
        sq_ref[...] = qs

    @pl.when(i != 0)
    def _acc():
        sum_ref[...] += ss
        sq_ref[...] += qs


def _mid_body(p_ref, y_ref, dinv_ref, b1_ref, w2_ref, out_ref):
    h1 = jax.nn.relu(dinv_ref[...] * (p_ref[0] + p_ref[1] + y_ref[...])
                     + b1_ref[...])
    out_ref[...] = jnp.dot(h1, w2_ref[...],
                           preferred_element_type=jnp.float32) * dinv_ref[...]


def _heads_body(p_ref, y_ref, dinv_ref, b2_ref, wt1_ref, bt1_ref,
                t_ref, sum_t_ref, sq_t_ref):
    h2 = dinv_ref[...] * (p_ref[0] + p_ref[1] + y_ref[...]) + b2_ref[...]
    t = jnp.dot(h2, wt1_ref[...], preferred_element_type=jnp.float32) \
        + bt1_ref[...]
    t_ref[...] = t
    st = jnp.sum(t, axis=0, keepdims=True)
    qt = jnp.sum(t * t, axis=0, keepdims=True)
    i = pl.program_id(0)

    @pl.when(i == 0)
    def _init():
        sum_t_ref[...] = st
        sq_t_ref[...] = qt

    @pl.when(i != 0)
    def _acc():
        sum_t_ref[...] += st
        sq_t_ref[...] += qt


def _final_body(t_ref, s_ref, st_ref, qt_ref, ss_ref, qs_ref,
                gt_ref, btb_ref, wt2_ref, bt2_ref,
                gs_ref, bsb_ref, ws2_ref, bs2_ref,
                zt_ref, zs_ref):
    inv_n = 1.0 / N

    def _head(x, sum_ref, sq_ref, g_ref, b_ref, w_ref, b2_ref, out_ref):
        m = sum_ref[...] * inv_n
        v = sq_ref[...] * inv_n - m * m
        xn = jax.nn.relu((x - m) * lax.rsqrt(v + 1e-5) * g_ref[...]
                         + b_ref[...])
        z = jnp.dot(xn, w_ref[...], preferred_element_type=jnp.float32) \
            + b2_ref[...]
        nrm = jnp.sqrt(jnp.sum(z * z, axis=1, keepdims=True))
        out_ref[...] = z / jnp.maximum(nrm, 1e-12)

    _head(t_ref[...], st_ref, qt_ref, gt_ref, btb_ref, wt2_ref, bt2_ref,
          zt_ref)
    _head(s_ref[...], ss_ref, qs_ref, gs_ref, bsb_ref, ws2_ref, bs2_ref,
          zs_ref)


def _full(shape):
    return pl.BlockSpec(shape, lambda i: (0,) * len(shape))


def _rows(shape):
    return pl.BlockSpec(shape, lambda i: (i,) + (0,) * (len(shape) - 1))


_f32 = jnp.float32


def _dinv_call(hists3d):
    return pl.pallas_call(
        _dinv_body,
        out_shape=jax.ShapeDtypeStruct((N_HIST // 128, 128), _f32),
    )(hists3d)


def _y1_call(x, w1, dinv_col):
    return pl.pallas_call(
        _y1_body,
        grid=(GRID_R,),
        in_specs=[_rows((RB, D)), _full((D, H)), _rows((RB, 1))],
        out_specs=_rows((RB, H)),
        out_shape=jax.ShapeDtypeStruct((N, H), _f32),
    )(x, w1, dinv_col)


def _sem_call(h_sem, ws1, bs1r):
    return pl.pallas_call(
        _sem_body,
        grid=(GRID_R,),
        in_specs=[_rows((RB, D)), _full((D, H)), _full((1, H))],
        out_specs=[_rows((RB, H)), _full((1, H)), _full((1, H))],
        out_shape=[jax.ShapeDtypeStruct((N, H), _f32),
                   jax.ShapeDtypeStruct((1, H), _f32),
                   jax.ShapeDtypeStruct((1, H), _f32)],
    )(h_sem, ws1, bs1r)


def _mid_call(p, y1, dinv_col, b1r, w2):
    return pl.pallas_call(
        _mid_body,
        grid=(GRID_R,),
        in_specs=[pl.BlockSpec((NC, RB, H), lambda i: (0, i, 0)),
                  _rows((RB, H)), _rows((RB, 1)), _full((1, H)),
                  _full((H, H))],
        out_specs=_rows((RB, H)),
        out_shape=jax.ShapeDtypeStruct((N, H), _f32),
    )(p, y1, dinv_col, b1r, w2)


def _heads_call(p, y2, dinv_col, b2r, wt1, bt1r):
    return pl.pallas_call(
        _heads_body,
        grid=(GRID_R,),
        in_specs=[pl.BlockSpec((NC, RB, H), lambda i: (0, i, 0)),
                  _rows((RB, H)), _rows((RB, 1)), _full((1, H)),
                  _full((H, H)), _full((1, H))],
        out_specs=[_rows((RB, H)),
                   _full((1, H)), _full((1, H))],
        out_shape=[jax.ShapeDtypeStruct((N, H), _f32),
                   jax.ShapeDtypeStruct((1, H), _f32),
                   jax.ShapeDtypeStruct((1, H), _f32)],
    )(p, y2, dinv_col, b2r, wt1, bt1r)


def _final_call(t, s, st, qt, ss, qs, gtr, btbr, wt2, bt2r,
                gsr, bsbr, ws2, bs2r):
    return pl.pallas_call(
        _final_body,
        grid=(GRID_R,),
        in_specs=[_rows((RB, H)), _rows((RB, H)),
                  _full((1, H)), _full((1, H)), _full((1, H)), _full((1, H)),
                  _full((1, H)), _full((1, H)), _full((H, Z)), _full((1, Z)),
                  _full((1, H)), _full((1, H)), _full((H, Z)), _full((1, Z))],
        out_specs=[_rows((RB, Z)), _rows((RB, Z))],
        out_shape=[jax.ShapeDtypeStruct((N, Z), _f32),
                   jax.ShapeDtypeStruct((N, Z), _f32)],
    )(t, s, st, qt, ss, qs, gtr, btbr, wt2, bt2r, gsr, bsbr, ws2, bs2r)


def kernel(x_topo, edge_index, h_sem, W1, b1, W2, b2, Wt1, bt1, gt, btb,
           Wt2, bt2, Ws1, bs1, gs, bsb, Ws2, bs2):
    src_idx = edge_index[0].reshape(NC, NS, NPAN, G, CL)
    dst_idx = edge_index[1].reshape(NC, NS, CHUNKS, CL)
    dst_flat = edge_index[1].reshape(NW, EPW)
    zero_slab = jnp.zeros((SLAB, D), _f32)

    hists = _deg_kernel()(dst_flat)
    dinv = _dinv_call(hists.reshape(NW, N_HIST // 128, 128))
    dinv_col = dinv.reshape(N_HIST, 1)[:N]

    b1r, b2r = b1.reshape(1, H), b2.reshape(1, H)
    bt1r, bs1r = bt1.reshape(1, H), bs1.reshape(1, H)
    gtr, btbr = gt.reshape(1, H), btb.reshape(1, H)
    gsr, bsbr = gs.reshape(1, H), bsb.reshape(1, H)
    bt2r, bs2r = bt2.reshape(1, Z), bs2.reshape(1, Z)

    y1 = _y1_call(x_topo, W1, dinv_col)
    p1 = _prop_kernel()(y1, src_idx, dst_idx, zero_slab)
    s, ss, qs = _sem_call(h_sem, Ws1, bs1r)
    y2 = _mid_call(p1, y1, dinv_col, b1r, W2)
    p2 = _prop_kernel()(y2, src_idx, dst_idx, zero_slab)
    t, st, qt = _heads_call(p2, y2, dinv_col, b2r, Wt1, bt1r)
    z_topo, z_sem = _final_call(t, s, st, qt, ss, qs, gtr, btbr, Wt2, bt2r,
                                gsr, bsbr, Ws2, bs2r)
    return (z_topo, z_sem)
